# baseline (speedup 1.0000x reference)
import jax
import jax.numpy as jnp
from jax.experimental import pallas as pl
from jax.experimental.pallas import tpu as pltpu

BLOCK_ROWS = 2048


def _pe_add_kernel(idx_ref, x_ref, s1_ref, c1_ref, s2_ref, c2_ref,
                   s3_ref, c3_ref, out_ref):
    d = x_ref.shape[1]
    idx = idx_ref[0, 0, :]
    idxb = jnp.broadcast_to(idx[:, None], (BLOCK_ROWS, d))
    u = idxb >> 6
    v = (idxb >> 3) & 7
    w = idxb & 7

    def gat(ref, ind):
        return jnp.take_along_axis(ref[...], ind, axis=0,
                                   mode="promise_in_bounds")

    s_a, c_a = gat(s1_ref, u), gat(c1_ref, u)
    s_b, c_b = gat(s2_ref, v), gat(c2_ref, v)
    s_c, c_c = gat(s3_ref, w), gat(c3_ref, w)
    s_bc = s_b * c_c + c_b * s_c
    c_bc = c_b * c_c - s_b * s_c
    pe = s_a * c_bc + c_a * s_bc
    out_ref[...] = x_ref[...] + pe


def _factor_tables(table):
    sin_cols = table[:, 0::2]
    cos_cols = table[:, 1::2]

    def dup(a):
        return jnp.repeat(a, 2, axis=1)

    s1 = dup(sin_cols[0::64])
    c1 = dup(cos_cols[0::64])
    s2 = dup(sin_cols[0::8][:8])
    c2 = dup(cos_cols[0::8][:8])
    w_sin = sin_cols[:8]
    w_cos = cos_cols[:8]
    s3 = jnp.stack([w_sin, w_cos], axis=2).reshape(8, -1)
    c3 = jnp.stack([w_cos, -w_sin], axis=2).reshape(8, -1)
    return s1, c1, s2, c2, s3, c3


def kernel(x, voxel_level, positional_encoding_table):
    b, s, d = x.shape
    n = b * s
    num_blocks = n // BLOCK_ROWS
    xf = x.reshape(n, d)
    idx = voxel_level.astype(jnp.int32).reshape(num_blocks, 1, BLOCK_ROWS)
    facs = _factor_tables(positional_encoding_table)

    small = pl.BlockSpec((8, d), lambda i: (0, 0))
    out = pl.pallas_call(
        _pe_add_kernel,
        grid=(num_blocks,),
        in_specs=[
            pl.BlockSpec((1, 1, BLOCK_ROWS), lambda i: (i, 0, 0)),
            pl.BlockSpec((BLOCK_ROWS, d), lambda i: (i, 0)),
            small, small, small, small, small, small,
        ],
        out_specs=pl.BlockSpec((BLOCK_ROWS, d), lambda i: (i, 0)),
        out_shape=jax.ShapeDtypeStruct((n, d), x.dtype),
        compiler_params=pltpu.CompilerParams(
            dimension_semantics=("arbitrary",),
        ),
    )(idx, xf, *facs)
    return out.reshape(b, s, d)
